# baseline (speedup 1.0000x reference)
import jax
import jax.numpy as jnp
from jax.experimental import pallas as pl
from jax.experimental.pallas import tpu as pltpu

S, D = 2048, 1024
P1, P2 = 256, 64


def _proj_body(x_ref, p_ref, o_ref):
    o_ref[...] = jnp.dot(
        x_ref[...], p_ref[...], preferred_element_type=jnp.float32
    ).astype(jnp.bfloat16)


def _matmul_body(a_ref, w_ref, b_ref, o_ref):
    acc = jnp.dot(a_ref[...], w_ref[...], preferred_element_type=jnp.float32)
    o_ref[...] = acc + b_ref[...]


def _tiled_matmul(a, w, b2d, tn):
    k = a.shape[1]
    n = w.shape[1]
    grid = pl.cdiv(n, tn)
    return pl.pallas_call(
        _matmul_body,
        grid=(grid,),
        in_specs=[
            pl.BlockSpec((S, k), lambda j: (0, 0)),
            pl.BlockSpec((k, tn), lambda j: (0, j)),
            pl.BlockSpec((1, tn), lambda j: (0, j)),
        ],
        out_specs=pl.BlockSpec((S, tn), lambda j: (0, j)),
        out_shape=jax.ShapeDtypeStruct((S, n), jnp.float32),
        compiler_params=pltpu.CompilerParams(
            dimension_semantics=("parallel",),
        ),
    )(a, w, b2d)


def kernel(x, proj1, proj2, W0, W1, W2, b0, b1, b2):
    x2 = x.reshape(S, D).astype(jnp.bfloat16)
    projc = jnp.concatenate([proj1, proj2], axis=1).astype(jnp.bfloat16)
    w0 = W0.astype(jnp.bfloat16)
    w1 = W1.astype(jnp.bfloat16)
    w2 = W2.astype(jnp.bfloat16)
    xp = pl.pallas_call(
        _proj_body,
        out_shape=jax.ShapeDtypeStruct((S, P1 + P2), jnp.bfloat16),
    )(x2, projc)
    xp1 = xp[:, :P1]
    xp2 = xp[:, P1:]

    head = _tiled_matmul(x2, w0, b0.reshape(1, -1), 1024)
    c1 = _tiled_matmul(xp1, w1, b1.reshape(1, -1), 2048)
    c2 = _tiled_matmul(xp2, w2, b2.reshape(1, -1), 2048)

    n0, n1, n2 = W0.shape[1], W1.shape[1], W2.shape[1]
    return (head.reshape(1, S, n0), c1.reshape(1, S, n1), c2.reshape(1, S, n2))


# speedup vs baseline: 1.7032x; 1.7032x over previous
import jax
import jax.numpy as jnp
from jax.experimental import pallas as pl
from jax.experimental.pallas import tpu as pltpu

S, D = 2048, 1024
P1, P2 = 256, 64


def _prep_body(x_ref, p_ref, xt_ref, xp1t_ref, xp2t_ref):
    xb = x_ref[...].astype(jnp.bfloat16)
    xt = xb.T
    xt_ref[...] = xt
    pb = p_ref[...].astype(jnp.bfloat16)
    xpt = jax.lax.dot_general(
        pb, xt, (((0,), (0,)), ((), ())),
        preferred_element_type=jnp.float32,
    ).astype(jnp.bfloat16)
    xp1t_ref[...] = xpt[:P1]
    xp2t_ref[...] = xpt[P1:]


def _tmatmul_body(w_ref, xt_ref, b_ref, o_ref):
    acc = jax.lax.dot_general(
        w_ref[...], xt_ref[...], (((0,), (0,)), ((), ())),
        preferred_element_type=jnp.float32,
    )
    o_ref[...] = acc + b_ref[...]


def _tiled_tmatmul(w, xt, bias_col, tn):
    k, n = w.shape
    grid = pl.cdiv(n, tn)
    out_t = pl.pallas_call(
        _tmatmul_body,
        grid=(grid,),
        in_specs=[
            pl.BlockSpec((k, tn), lambda j: (0, j)),
            pl.BlockSpec((k, S), lambda j: (0, 0)),
            pl.BlockSpec((tn, 1), lambda j: (j, 0)),
        ],
        out_specs=pl.BlockSpec((tn, S), lambda j: (j, 0)),
        out_shape=jax.ShapeDtypeStruct((n, S), jnp.float32),
        compiler_params=pltpu.CompilerParams(
            dimension_semantics=("parallel",),
        ),
    )(w, xt, bias_col)
    return out_t.T.reshape(1, S, n)


def kernel(x, proj1, proj2, W0, W1, W2, b0, b1, b2):
    x2 = x.reshape(S, D)
    projc = jnp.concatenate([proj1, proj2], axis=1)
    w0 = W0.astype(jnp.bfloat16)
    w1 = W1.astype(jnp.bfloat16)
    w2 = W2.astype(jnp.bfloat16)
    xt, xp1t, xp2t = pl.pallas_call(
        _prep_body,
        out_shape=(
            jax.ShapeDtypeStruct((D, S), jnp.bfloat16),
            jax.ShapeDtypeStruct((P1, S), jnp.bfloat16),
            jax.ShapeDtypeStruct((P2, S), jnp.bfloat16),
        ),
    )(x2, projc)

    head = _tiled_tmatmul(w0, xt, b0.reshape(-1, 1), 1024)
    c1 = _tiled_tmatmul(w1, xp1t, b1.reshape(-1, 1), 2048)
    c2 = _tiled_tmatmul(w2, xp2t, b2.reshape(-1, 1), 2048)
    return (head, c1, c2)


# speedup vs baseline: 2.2351x; 1.3123x over previous
import jax
import jax.numpy as jnp
from jax.experimental import pallas as pl
from jax.experimental.pallas import tpu as pltpu

S, D = 2048, 1024
P1, P2 = 256, 64


def _prep_body(x_ref, p_ref, xt_ref, xp1t_ref, xp2t_ref):
    xb = x_ref[...].astype(jnp.bfloat16)
    xt = xb.T
    xt_ref[...] = xt
    pb = p_ref[...].astype(jnp.bfloat16)
    xpt = jax.lax.dot_general(
        pb, xt, (((0,), (0,)), ((), ())),
        preferred_element_type=jnp.float32,
    ).astype(jnp.bfloat16)
    xp1t_ref[...] = xpt[:P1]
    xp2t_ref[...] = xpt[P1:]


def _tmatmul_body(wt_ref, xt_ref, b_ref, o_ref):
    wb = wt_ref[...].astype(jnp.bfloat16)
    acc = jnp.dot(wb, xt_ref[...], preferred_element_type=jnp.float32)
    o_ref[...] = acc + b_ref[...].T


def _tmatmul_tlhs_body(w_ref, xt_ref, b_ref, o_ref):
    wb = w_ref[...].astype(jnp.bfloat16)
    acc = jax.lax.dot_general(
        wb, xt_ref[...], (((0,), (0,)), ((), ())),
        preferred_element_type=jnp.float32,
    )
    o_ref[...] = acc + b_ref[...].T


def _tiled_tmatmul_tlhs(w, xt, bias_row, tn):
    k, n = w.shape
    grid = pl.cdiv(n, tn)
    out_t = pl.pallas_call(
        _tmatmul_tlhs_body,
        grid=(grid,),
        in_specs=[
            pl.BlockSpec((k, tn), lambda j: (0, j)),
            pl.BlockSpec((k, S), lambda j: (0, 0)),
            pl.BlockSpec((1, tn), lambda j: (0, j)),
        ],
        out_specs=pl.BlockSpec((tn, S), lambda j: (j, 0)),
        out_shape=jax.ShapeDtypeStruct((n, S), jnp.float32),
        compiler_params=pltpu.CompilerParams(
            dimension_semantics=("parallel",),
        ),
    )(w, xt, bias_row)
    return out_t.T.reshape(1, S, n)


def _tiled_tmatmul(wt, xt, bias_row, tn):
    n, k = wt.shape
    grid = pl.cdiv(n, tn)
    out_t = pl.pallas_call(
        _tmatmul_body,
        grid=(grid,),
        in_specs=[
            pl.BlockSpec((tn, k), lambda j: (j, 0)),
            pl.BlockSpec((k, S), lambda j: (0, 0)),
            pl.BlockSpec((1, tn), lambda j: (0, j)),
        ],
        out_specs=pl.BlockSpec((tn, S), lambda j: (j, 0)),
        out_shape=jax.ShapeDtypeStruct((n, S), jnp.float32),
        compiler_params=pltpu.CompilerParams(
            dimension_semantics=("parallel",),
        ),
    )(wt, xt, bias_row)
    return out_t.T.reshape(1, S, n)


def kernel(x, proj1, proj2, W0, W1, W2, b0, b1, b2):
    x2 = x.reshape(S, D)
    projc = jnp.concatenate([proj1, proj2], axis=1)
    xt, xp1t, xp2t = pl.pallas_call(
        _prep_body,
        out_shape=(
            jax.ShapeDtypeStruct((D, S), jnp.bfloat16),
            jax.ShapeDtypeStruct((P1, S), jnp.bfloat16),
            jax.ShapeDtypeStruct((P2, S), jnp.bfloat16),
        ),
    )(x2, projc)

    head = _tiled_tmatmul(W0.T, xt, b0.reshape(1, -1), 1024)
    c1 = _tiled_tmatmul(W1.T, xp1t, b1.reshape(1, -1), 2048)
    c2 = _tiled_tmatmul_tlhs(W2, xp2t, b2.reshape(1, -1), 2048)
    return (head, c1, c2)
